# baseline (speedup 1.0000x reference)
import functools

import jax
import jax.numpy as jnp
from jax import lax
from jax.experimental import pallas as pl
from jax.experimental.pallas import tpu as pltpu
from jax.experimental.pallas import tpu_sc as plsc

_N, _T, _F = 1024, 512, 256
_B = 8
_NC, _NS = 2, 16
_NW = _NC * _NS
_NODES_PER_W = _N // _NW
_CH = 128
_NV = _F // 16


def _sc_body(x_hbm, meta_hbm, q_hbm, out_hbm, xb0, xb1, qv, ov, mv,
             sem0, sem1):
    wid = lax.axis_index("s") * _NC + lax.axis_index("c")
    pltpu.sync_copy(meta_hbm, mv)
    pltpu.sync_copy(q_hbm, qv)
    lane = lax.iota(jnp.int32, 16)
    dnums = lax.GatherDimensionNumbers(
        offset_dims=(), collapsed_slice_dims=(0,), start_index_map=(0,))

    def per_node(i, carry):
        node = i * _NW + wid
        ptrv = mv[pl.ds(0, 16)]
        lenv = mv[pl.ds(16, 16)]
        gv = jnp.where((lane >= 1) & (lane <= _B - 1) & (ptrv <= node), 1, 0)
        g = jnp.sum(gv)
        node_len = jnp.sum(jnp.where(lane == g, lenv, 0))
        nch = (node_len + _CH - 1) // _CH

        pltpu.async_copy(x_hbm.at[node, pl.ds(0, _CH), :], xb0, sem0)

        @pl.when(nch > 1)
        def _prime1():
            pltpu.async_copy(x_hbm.at[node, pl.ds(_CH, _CH), :], xb1, sem1)

        zero = jnp.zeros((16,), jnp.float32)
        for j in range(_NV):
            ov[pl.ds(16 * j, 16)] = zero

        def outer(k, qs):
            for b, (buf, sem) in enumerate(((xb0, sem0), (xb1, sem1))):
                c = 2 * k + b

                @pl.when(c < nch)
                def _wait():
                    pltpu.make_async_copy(
                        x_hbm.at[node, pl.ds(0, _CH), :], buf, sem).wait()

                rows = jnp.maximum(
                    0, jnp.minimum(_CH, node_len - c * _CH))

                def per_row(t, qs, buf=buf):
                    xs = [buf[t, pl.ds(16 * j, 16)] for j in range(_NV)]
                    ps = [xs[j] * qs[j] for j in range(_NV)]
                    while len(ps) > 1:
                        ps = [ps[m] + ps[m + 1] for m in range(0, len(ps), 2)]
                    d = ps[0]
                    for shift in (8, 4, 2, 1):
                        idx = jnp.bitwise_xor(lane, shift)
                        d = d + lax.gather(
                            d, idx[:, None], dnums, (1,),
                            mode=lax.GatherScatterMode.PROMISE_IN_BOUNDS)
                    for j in range(_NV):
                        plsc.addupdate(ov.at[pl.ds(16 * j, 16)], xs[j] * d)
                    return qs

                qs = plsc.parallel_loop(0, rows, 1, unroll=8, carry=qs)(per_row)

                @pl.when(c + 2 < nch)
                def _start_next():
                    pltpu.async_copy(
                        x_hbm.at[node, pl.ds((c + 2) * _CH, _CH), :], buf, sem)
            return qs

        qs = tuple(qv[pl.ds(16 * j, 16)] for j in range(_NV))
        lax.fori_loop(0, (nch + 1) // 2, outer, qs)
        pltpu.sync_copy(ov, out_hbm.at[node])
        return carry

    lax.fori_loop(0, _NODES_PER_W, per_node, jnp.int32(0))


def kernel(nodes_output, ptr, lengths, Wq_w):
    ptr_i = ptr.astype(jnp.int32)
    len_i = lengths.astype(jnp.int32)
    meta = jnp.full((32,), _N + _T, jnp.int32)
    meta = meta.at[0:_B + 1].set(ptr_i).at[16:16 + _B].set(len_i)

    mesh = plsc.VectorSubcoreMesh(core_axis_name="c", subcore_axis_name="s")
    run = functools.partial(
        pl.kernel,
        mesh=mesh,
        out_type=jax.ShapeDtypeStruct((_N, _F), jnp.float32),
        compiler_params=pltpu.CompilerParams(needs_layout_passes=False),
        scratch_types=[
            pltpu.VMEM((_CH, _F), jnp.float32),
            pltpu.VMEM((_CH, _F), jnp.float32),
            pltpu.VMEM((_F,), jnp.float32),
            pltpu.VMEM((_F,), jnp.float32),
            pltpu.VMEM((32,), jnp.int32),
            pltpu.SemaphoreType.DMA,
            pltpu.SemaphoreType.DMA,
        ],
    )(_sc_body)
    return run(nodes_output, meta, Wq_w)


# speedup vs baseline: 2.1245x; 2.1245x over previous
import functools

import jax
import jax.numpy as jnp
from jax import lax
from jax.experimental import pallas as pl
from jax.experimental.pallas import tpu as pltpu
from jax.experimental.pallas import tpu_sc as plsc

_N, _T, _F = 1024, 512, 256
_B = 8
_S = 384

_NC, _NS = 2, 16
_NW = _NC * _NS
_SC_NODES_PER_W = _S // _NW
_CH = 64
_NV = _F // 16

_NB = 64
_TB = 64
_NT = _T // _TB
_TC_N = _N - _S
_GRP_OFF = _S // _NB


def _sc_body(x_hbm, meta_hbm, q_hbm, out_hbm, xb0, xb1, qv, ov, mv,
             sem0, sem1):
    wid = lax.axis_index("s") * _NC + lax.axis_index("c")
    pltpu.sync_copy(meta_hbm, mv)
    pltpu.sync_copy(q_hbm, qv)
    lane = lax.iota(jnp.int32, 16)
    dnums = lax.GatherDimensionNumbers(
        offset_dims=(), collapsed_slice_dims=(0,), start_index_map=(0,))

    def per_node(i, carry):
        node = i * _NW + wid
        ptrv = mv[pl.ds(0, 16)]
        lenv = mv[pl.ds(16, 16)]
        gv = jnp.where((lane >= 1) & (lane <= _B - 1) & (ptrv <= node), 1, 0)
        g = jnp.sum(gv)
        node_len = jnp.sum(jnp.where(lane == g, lenv, 0))
        nch = (node_len + _CH - 1) // _CH

        pltpu.async_copy(x_hbm.at[node, pl.ds(0, _CH), :], xb0, sem0)

        @pl.when(nch > 1)
        def _prime1():
            pltpu.async_copy(x_hbm.at[node, pl.ds(_CH, _CH), :], xb1, sem1)

        zero = jnp.zeros((16,), jnp.float32)
        for j in range(_NV):
            ov[pl.ds(16 * j, 16)] = zero

        def outer(k, qs):
            for b, (buf, sem) in enumerate(((xb0, sem0), (xb1, sem1))):
                c = 2 * k + b

                @pl.when(c < nch)
                def _wait():
                    pltpu.make_async_copy(
                        x_hbm.at[node, pl.ds(0, _CH), :], buf, sem).wait()

                rows = jnp.maximum(
                    0, jnp.minimum(_CH, node_len - c * _CH))

                def per_row(t, qs, buf=buf):
                    xs = [buf[t, pl.ds(16 * j, 16)] for j in range(_NV)]
                    ps = [xs[j] * qs[j] for j in range(_NV)]
                    while len(ps) > 1:
                        ps = [ps[m] + ps[m + 1] for m in range(0, len(ps), 2)]
                    d = ps[0]
                    for shift in (8, 4, 2, 1):
                        idx = jnp.bitwise_xor(lane, shift)
                        d = d + lax.gather(
                            d, idx[:, None], dnums, (1,),
                            mode=lax.GatherScatterMode.PROMISE_IN_BOUNDS)
                    for j in range(_NV):
                        plsc.addupdate(ov.at[pl.ds(16 * j, 16)], xs[j] * d)
                    return qs

                qs = plsc.parallel_loop(0, rows, 1, unroll=4, carry=qs)(per_row)

                @pl.when(c + 2 < nch)
                def _start_next():
                    pltpu.async_copy(
                        x_hbm.at[node, pl.ds((c + 2) * _CH, _CH), :], buf, sem)
            return qs

        qs = tuple(qv[pl.ds(16 * j, 16)] for j in range(_NV))
        lax.fori_loop(0, (nch + 1) // 2, outer, qs)
        pltpu.sync_copy(ov, out_hbm.at[node])
        return carry

    lax.fori_loop(0, _SC_NODES_PER_W, per_node, jnp.int32(0))


def _sc_run(nodes_output, meta, Wq_w):
    mesh = plsc.VectorSubcoreMesh(core_axis_name="c", subcore_axis_name="s")
    run = functools.partial(
        pl.kernel,
        mesh=mesh,
        out_type=jax.ShapeDtypeStruct((_S, _F), jnp.float32),
        compiler_params=pltpu.CompilerParams(needs_layout_passes=False),
        scratch_types=[
            pltpu.VMEM((_CH, _F), jnp.float32),
            pltpu.VMEM((_CH, _F), jnp.float32),
            pltpu.VMEM((_F,), jnp.float32),
            pltpu.VMEM((_F,), jnp.float32),
            pltpu.VMEM((32,), jnp.int32),
            pltpu.SemaphoreType.DMA,
            pltpu.SemaphoreType.DMA,
        ],
    )(_sc_body)
    return run(nodes_output, meta, Wq_w)


def _tc_body(nlen_ref, nblk_ref, x_ref, q_ref, o_ref):
    n = pl.program_id(0)
    t = pl.program_id(1)

    @pl.when(t == 0)
    def _init():
        o_ref[...] = jnp.zeros_like(o_ref)

    @pl.when(t < nblk_ref[n])
    def _step():
        q = q_ref[0]
        x2 = x_ref[...].reshape(_NB * _TB, _F)
        s = jax.lax.dot_general(
            x2, q.reshape(_F, 1),
            dimension_numbers=(((1,), (0,)), ((), ())),
            preferred_element_type=jnp.float32,
        )
        sg = 8
        cols = sg * _TB
        col = lax.broadcasted_iota(jnp.int32, (sg, cols), 1)
        row = lax.broadcasted_iota(jnp.int32, (sg, cols), 0)
        diag = col // _TB == row
        t_loc = (col - row * _TB) + t * _TB
        s2 = s.reshape(_NB // sg, 1, cols)
        for j in range(_NB // sg):
            lens = jnp.concatenate(
                [jnp.full((1, cols), nlen_ref[n * _NB + j * sg + k], jnp.int32)
                 for k in range(sg)], axis=0)
            w_bd = jnp.where(diag & (t_loc < lens), s2[j], 0.0)
            o_ref[j * sg:(j + 1) * sg, :] += jax.lax.dot_general(
                w_bd, x_ref[j * sg:(j + 1) * sg].reshape(cols, _F),
                dimension_numbers=(((1,), (0,)), ((), ())),
                preferred_element_type=jnp.float32,
            )


def _tc_run(nodes_output, node_len_tc, grp_nblk, q2):
    grid_spec = pltpu.PrefetchScalarGridSpec(
        num_scalar_prefetch=2,
        grid=(_TC_N // _NB, _NT),
        in_specs=[
            pl.BlockSpec(
                (_NB, _TB, _F),
                lambda n, t, nlen, nblk: (
                    n + _GRP_OFF, jnp.minimum(t, nblk[n] - 1), 0),
            ),
            pl.BlockSpec((1, _F), lambda n, t, nlen, nblk: (0, 0)),
        ],
        out_specs=pl.BlockSpec((_NB, _F), lambda n, t, nlen, nblk: (n, 0)),
    )
    return pl.pallas_call(
        _tc_body,
        grid_spec=grid_spec,
        out_shape=jax.ShapeDtypeStruct((_TC_N, _F), jnp.float32),
    )(node_len_tc, grp_nblk, nodes_output, q2)


def kernel(nodes_output, ptr, lengths, Wq_w):
    ptr_i = ptr.astype(jnp.int32)
    len_i = lengths.astype(jnp.int32)

    meta = jnp.full((32,), _N + _T, jnp.int32)
    meta = meta.at[0:_B + 1].set(ptr_i).at[16:16 + _B].set(len_i)

    num_nodes = ptr_i[1:] - ptr_i[:-1]
    node_len = jnp.repeat(len_i, num_nodes, total_repeat_length=_N)
    node_len_tc = node_len[_S:]
    grp_max = jnp.max(node_len_tc.reshape(_TC_N // _NB, _NB), axis=1)
    grp_nblk = (grp_max + (_TB - 1)) // _TB
    q2 = Wq_w.reshape(1, _F)

    sc_out = _sc_run(nodes_output, meta, Wq_w)
    tc_out = _tc_run(nodes_output, node_len_tc, grp_nblk, q2)
    return jnp.concatenate([sc_out, tc_out], axis=0)
